# baseline (speedup 1.0000x reference)
import functools

import jax
import jax.numpy as jnp
from jax import lax
from jax.experimental import pallas as pl
from jax.experimental.pallas import tpu as pltpu
from jax.experimental.pallas import tpu_sc as plsc

_B, _C, _H, _W = 16, 6, 512, 512
_HW = _H * _W
_L = 16
_NW = 32
_PXB = _HW // _NW
_CH = 2048
_NCH = _PXB // _CH
_STEPS = _CH // _L


def _sc_body(x_ref, t_ref, wsum_ref, bits_ref, buf, wout, bout, sem0, sem1):
    cid = lax.axis_index("c")
    sid = lax.axis_index("s")
    wid = cid * 16 + sid
    px0 = wid * _PXB

    def fire(k, ch, par, sem):
        base = px0 + ch * _CH
        pltpu.make_async_copy(
            x_ref.at[k, :, pl.ds(base, _CH)], buf.at[par, 0], sem).start()
        pltpu.make_async_copy(
            t_ref.at[k, :, pl.ds(base, _CH)], buf.at[par, 1], sem).start()

    def drain(par, sem):
        pltpu.make_async_copy(
            x_ref.at[0, :, pl.ds(0, _CH)], buf.at[par, 0], sem).wait()
        pltpu.make_async_copy(
            t_ref.at[0, :, pl.ds(0, _CH)], buf.at[par, 1], sem).wait()

    def chunk_accum(par, carry):
        def step(i, c2):
            aw, ab = c2
            base = i * _L
            xs = [buf[par, 0, c, pl.ds(base, _L)] for c in range(_C)]
            ts = [buf[par, 1, c, pl.ds(base, _L)] for c in range(_C)]
            s = xs[0]
            q = xs[0] * xs[0]
            for c in range(1, _C):
                s = s + xs[c]
                q = q + xs[c] * xs[c]
            w = q - s * s * (1.0 / _C)
            m = ts[0]
            bit = jnp.full((_L,), 1, jnp.int32)
            for c in range(1, _C):
                gt = ts[c] > m
                m = jnp.where(gt, ts[c], m)
                bit = jnp.where(gt, jnp.int32(1 << c), bit)
            aw = aw + jnp.where(bit > 1, w, 0.0)
            ab = ab | bit
            return aw, ab

        return lax.fori_loop(0, _STEPS, step, carry)

    sems = (sem0, sem1)
    fire(0, 0, 0, sem0)

    def batch_body(k, _):
        acc = (jnp.zeros((_L,), jnp.float32), jnp.zeros((_L,), jnp.int32))
        for ch in range(_NCH):
            nxt = ch + 1
            if nxt < _NCH:
                fire(k, nxt, nxt % 2, sems[nxt % 2])
            else:
                @pl.when(k + 1 < _B)
                def _():
                    fire(k + 1, 0, 0, sem0)

            par = ch % 2
            drain(par, sems[par])
            acc = chunk_accum(par, acc)
        wout[pl.ds(k * _L, _L)] = acc[0]
        bout[pl.ds(k * _L, _L)] = acc[1]
        return _

    lax.fori_loop(0, _B, batch_body, None)
    pltpu.sync_copy(wout, wsum_ref.at[wid])
    pltpu.sync_copy(bout, bits_ref.at[wid])


@functools.partial(
    pl.kernel,
    mesh=plsc.VectorSubcoreMesh(core_axis_name="c", subcore_axis_name="s"),
    out_type=[
        jax.ShapeDtypeStruct((_NW, _B * _L), jnp.float32),
        jax.ShapeDtypeStruct((_NW, _B * _L), jnp.int32),
    ],
    scratch_types=[
        pltpu.VMEM((2, 2, _C, _CH), jnp.float32),
        pltpu.VMEM((_B * _L,), jnp.float32),
        pltpu.VMEM((_B * _L,), jnp.int32),
        pltpu.SemaphoreType.DMA,
        pltpu.SemaphoreType.DMA,
    ],
)
def _sc_partials(x_ref, t_ref, wsum_ref, bits_ref, buf, wout, bout, s0, s1):
    _sc_body(x_ref, t_ref, wsum_ref, bits_ref, buf, wout, bout, s0, s1)


@jax.jit
def kernel(input, target):
    x3 = input.reshape(_B, _C, _HW)
    t3 = target.reshape(_B, _C, _HW)
    wsum, bits = _sc_partials(x3, t3)
    w = wsum.reshape(_NW, _B, _L).sum(axis=(0, 2)) * (1.0 / (_C - 1))
    bb = bits.reshape(_NW, _B, _L)
    n_uniq = jnp.zeros((_B,), jnp.float32)
    for c in range(1, _C):
        present = jnp.any((bb & (1 << c)) != 0, axis=(0, 2))
        n_uniq = n_uniq + present.astype(jnp.float32)
    return jnp.mean(w / (n_uniq + 1e-8))
